# Initial kernel scaffold; baseline (speedup 1.0000x reference)
import functools

import jax
import jax.numpy as jnp
from jax import lax
from jax.experimental import pallas as pl
from jax.experimental.pallas import tpu as pltpu
from jax.experimental.pallas import tpu_sc as plsc

_VOCAB = 1000000
_D = 32
_H = 64
_C = 3
_B = 16384
_L = 50

_NC = 2
_NS = 16
_NW = _NC * _NS

_CB = 64
_CHUNK = 128
_IDX_PER_STEP = _CB * _L
_NCHUNK = _IDX_PER_STEP // _CHUNK
_ROWS_PER_W = _B // _NW
_NSTEP = _ROWS_PER_W // _CB


def _pool_body(emb_hbm, ids_hbm, out_hbm, idx_v, rows_v, acc_v, sem):
    wid = lax.axis_index("s") * _NC + lax.axis_index("c")
    base_row = wid * _ROWS_PER_W

    def step(s, carry):
        row0 = base_row + s * _CB
        pltpu.sync_copy(ids_hbm.at[pl.ds(row0 * _L, _IDX_PER_STEP)], idx_v)

        def fire(c, carry2):
            off = c * _CHUNK
            pltpu.async_copy(
                emb_hbm.at[idx_v.at[pl.ds(off, _CHUNK)]],
                rows_v.at[pl.ds(off, _CHUNK)],
                sem,
            )
            return carry2

        lax.fori_loop(0, _NCHUNK, fire, 0)
        pltpu.make_async_copy(
            emb_hbm.at[pl.ds(0, _IDX_PER_STEP)], rows_v, sem
        ).wait()

        inv = jnp.float32(1.0 / _L)

        def pool_one(i, carry3):
            j0 = i * _L

            def add_tok(l, acc):
                j = j0 + l
                return (acc[0] + rows_v[j, pl.ds(0, 16)],
                        acc[1] + rows_v[j, pl.ds(16, 16)])

            a0, a1 = lax.fori_loop(
                0, _L, add_tok,
                (jnp.zeros((16,), jnp.float32), jnp.zeros((16,), jnp.float32)),
                unroll=True,
            )
            acc_v[i, pl.ds(0, 16)] = a0 * inv
            acc_v[i, pl.ds(16, 16)] = a1 * inv
            return carry3

        lax.fori_loop(0, _CB, pool_one, 0)
        pltpu.sync_copy(acc_v, out_hbm.at[pl.ds(row0, _CB)])
        return carry

    lax.fori_loop(0, _NSTEP, step, 0)


@jax.jit
def _pool(ids_flat, emb):
    mesh = plsc.VectorSubcoreMesh(core_axis_name="c", subcore_axis_name="s")
    return pl.kernel(
        _pool_body,
        out_type=jax.ShapeDtypeStruct((_B, _D), jnp.float32),
        mesh=mesh,
        scratch_types=[
            pltpu.VMEM((_IDX_PER_STEP,), jnp.int32),
            pltpu.VMEM((_IDX_PER_STEP, _D), jnp.float32),
            pltpu.VMEM((_CB, _D), jnp.float32),
            pltpu.SemaphoreType.DMA,
        ],
    )(emb, ids_flat)


def _mlp_body(x_ref, w1_ref, b1_ref, w2_ref, b2_ref, o_ref):
    x = x_ref[...]
    h = jnp.dot(x, w1_ref[...], preferred_element_type=jnp.float32)
    h = jnp.maximum(h + b1_ref[...], 0.0)
    o_ref[...] = (
        jnp.dot(h, w2_ref[...], preferred_element_type=jnp.float32)
        + b2_ref[...]
    )


@jax.jit
def _mlp(x, W1, b1, W2, b2):
    blk = 2048
    grid = _B // blk
    return pl.pallas_call(
        _mlp_body,
        grid=(grid,),
        in_specs=[
            pl.BlockSpec((blk, _D), lambda i: (i, 0)),
            pl.BlockSpec((_D, _H), lambda i: (0, 0)),
            pl.BlockSpec((1, _H), lambda i: (0, 0)),
            pl.BlockSpec((_H, _C), lambda i: (0, 0)),
            pl.BlockSpec((1, _C), lambda i: (0, 0)),
        ],
        out_specs=pl.BlockSpec((blk, _C), lambda i: (i, 0)),
        out_shape=jax.ShapeDtypeStruct((_B, _C), jnp.float32),
    )(x, W1, b1.reshape(1, _H), W2, b2.reshape(1, _C))


def kernel(ids, emb, W1, b1, W2, b2):
    ids_flat = ids.reshape(-1).astype(jnp.int32)
    pooled = _pool(ids_flat, emb)
    return _mlp(pooled, W1, b1, W2, b2)


# speedup vs baseline: 2.7533x; 2.7533x over previous
import functools

import jax
import jax.numpy as jnp
from jax import lax
from jax.experimental import pallas as pl
from jax.experimental.pallas import tpu as pltpu
from jax.experimental.pallas import tpu_sc as plsc

_VOCAB = 1000000
_D = 32
_H = 64
_C = 3
_B = 16384
_L = 50

_NC = 2
_NS = 16
_NW = _NC * _NS

_CB = 64
_CHUNK = 128
_IDX_PER_STEP = _CB * _L
_NCHUNK = _IDX_PER_STEP // _CHUNK
_ROWS_PER_W = _B // _NW
_NSTEP = _ROWS_PER_W // _CB


def _pool_body(emb_hbm, ids_hbm, out_hbm, idx_v, rows_v, acc_v, sem):
    wid = lax.axis_index("s") * _NC + lax.axis_index("c")
    base_row = wid * _ROWS_PER_W

    def step(s, carry):
        row0 = base_row + s * _CB
        pltpu.sync_copy(ids_hbm.at[pl.ds(row0 * _L, _IDX_PER_STEP)], idx_v)

        def fire(c, carry2):
            off = c * _CHUNK
            pltpu.async_copy(
                emb_hbm.at[idx_v.at[pl.ds(off, _CHUNK)]],
                rows_v.at[pl.ds(off, _CHUNK)],
                sem,
            )
            return carry2

        lax.fori_loop(0, _NCHUNK, fire, 0)
        pltpu.make_async_copy(
            emb_hbm.at[pl.ds(0, _IDX_PER_STEP)], rows_v, sem
        ).wait()

        inv = jnp.float32(1.0 / _L)

        def pool_one(i, carry3):
            j0 = i * _L

            def add_tok(l, acc):
                j = j0 + l
                return (acc[0] + rows_v[j, pl.ds(0, 16)],
                        acc[1] + rows_v[j, pl.ds(16, 16)])

            a0, a1 = lax.fori_loop(
                0, _L, add_tok,
                (jnp.zeros((16,), jnp.float32), jnp.zeros((16,), jnp.float32)),
                unroll=True,
            )
            acc_v[i, pl.ds(0, 16)] = a0 * inv
            acc_v[i, pl.ds(16, 16)] = a1 * inv
            return carry3

        lax.fori_loop(0, _CB, pool_one, 0)
        pltpu.sync_copy(acc_v, out_hbm.at[pl.ds(row0, _CB)])
        return carry

    lax.fori_loop(0, _NSTEP, step, 0)


@jax.jit
def _pool(ids_flat, emb):
    mesh = plsc.VectorSubcoreMesh(core_axis_name="c", subcore_axis_name="s")
    return pl.kernel(
        _pool_body,
        out_type=jax.ShapeDtypeStruct((_B, _D), jnp.float32),
        mesh=mesh,
        scratch_types=[
            pltpu.VMEM((_IDX_PER_STEP,), jnp.int32),
            pltpu.VMEM((_IDX_PER_STEP, _D), jnp.float32),
            pltpu.VMEM((_CB, _D), jnp.float32),
            pltpu.SemaphoreType.DMA,
        ],
        compiler_params=pltpu.CompilerParams(use_tc_tiling_on_sc=False),
    )(emb, ids_flat)


def _mlp_body(x_ref, w1_ref, b1_ref, w2_ref, b2_ref, o_ref):
    x = x_ref[...]
    h = jnp.dot(x, w1_ref[...], preferred_element_type=jnp.float32)
    h = jnp.maximum(h + b1_ref[...], 0.0)
    o_ref[...] = (
        jnp.dot(h, w2_ref[...], preferred_element_type=jnp.float32)
        + b2_ref[...]
    )


@jax.jit
def _mlp(x, W1, b1, W2, b2):
    blk = 2048
    grid = _B // blk
    return pl.pallas_call(
        _mlp_body,
        grid=(grid,),
        in_specs=[
            pl.BlockSpec((blk, _D), lambda i: (i, 0)),
            pl.BlockSpec((_D, _H), lambda i: (0, 0)),
            pl.BlockSpec((1, _H), lambda i: (0, 0)),
            pl.BlockSpec((_H, _C), lambda i: (0, 0)),
            pl.BlockSpec((1, _C), lambda i: (0, 0)),
        ],
        out_specs=pl.BlockSpec((blk, _C), lambda i: (i, 0)),
        out_shape=jax.ShapeDtypeStruct((_B, _C), jnp.float32),
    )(x, W1, b1.reshape(1, _H), W2, b2.reshape(1, _C))


def kernel(ids, emb, W1, b1, W2, b2):
    ids_flat = ids.reshape(-1).astype(jnp.int32)
    pooled = _pool(ids_flat, emb)
    return _mlp(pooled, W1, b1, W2, b2)


# speedup vs baseline: 2.8660x; 1.0409x over previous
import functools

import jax
import jax.numpy as jnp
from jax import lax
from jax.experimental import pallas as pl
from jax.experimental.pallas import tpu as pltpu
from jax.experimental.pallas import tpu_sc as plsc

_VOCAB = 1000000
_D = 32
_H = 64
_C = 3
_B = 16384
_L = 50

_NC = 2
_NS = 16
_NW = _NC * _NS

_CB = 32
_CHUNK = 80
_IDX_PER_STEP = _CB * _L
_NCHUNK = _IDX_PER_STEP // _CHUNK
_ROWS_PER_W = _B // _NW
_NSTEP = _ROWS_PER_W // _CB


def _pool_body(emb_hbm, ids_hbm, out_hbm,
               idx0, idx1, rows0, rows1, acc_v, sem0, sem1):
    wid = lax.axis_index("s") * _NC + lax.axis_index("c")
    base_row = wid * _ROWS_PER_W
    idx_bufs = (idx0, idx1)
    rows_bufs = (rows0, rows1)
    sems = (sem0, sem1)

    def stage_and_fire(s, p):
        row0 = base_row + s * _CB
        pltpu.sync_copy(ids_hbm.at[pl.ds(row0 * _L, _IDX_PER_STEP)],
                        idx_bufs[p])
        for c in range(_NCHUNK):
            off = c * _CHUNK
            pltpu.async_copy(
                emb_hbm.at[idx_bufs[p].at[pl.ds(off, _CHUNK)]],
                rows_bufs[p].at[pl.ds(off, _CHUNK)],
                sems[p],
            )

    inv = jnp.float32(1.0 / _L)
    stage_and_fire(0, 0)
    for s in range(_NSTEP):
        p = s % 2
        if s + 1 < _NSTEP:
            stage_and_fire(s + 1, (s + 1) % 2)
        pltpu.make_async_copy(
            emb_hbm.at[pl.ds(0, _IDX_PER_STEP)], rows_bufs[p], sems[p]
        ).wait()

        rows_v = rows_bufs[p]

        def pool_one(i, carry):
            j0 = i * _L

            def add_tok(l, acc):
                j = j0 + l
                return (acc[0] + rows_v[j, pl.ds(0, 16)],
                        acc[1] + rows_v[j, pl.ds(16, 16)])

            a0, a1 = lax.fori_loop(
                0, _L, add_tok,
                (jnp.zeros((16,), jnp.float32), jnp.zeros((16,), jnp.float32)),
                unroll=True,
            )
            acc_v[i, pl.ds(0, 16)] = a0 * inv
            acc_v[i, pl.ds(16, 16)] = a1 * inv
            return carry

        lax.fori_loop(0, _CB, pool_one, 0)
        pltpu.sync_copy(acc_v, out_hbm.at[pl.ds(base_row + s * _CB, _CB)])


@jax.jit
def _pool(ids_flat, emb):
    mesh = plsc.VectorSubcoreMesh(core_axis_name="c", subcore_axis_name="s")
    return pl.kernel(
        _pool_body,
        out_type=jax.ShapeDtypeStruct((_B, _D), jnp.float32),
        mesh=mesh,
        scratch_types=[
            pltpu.VMEM((_IDX_PER_STEP,), jnp.int32),
            pltpu.VMEM((_IDX_PER_STEP,), jnp.int32),
            pltpu.VMEM((_IDX_PER_STEP, _D), jnp.float32),
            pltpu.VMEM((_IDX_PER_STEP, _D), jnp.float32),
            pltpu.VMEM((_CB, _D), jnp.float32),
            pltpu.SemaphoreType.DMA,
            pltpu.SemaphoreType.DMA,
        ],
        compiler_params=pltpu.CompilerParams(use_tc_tiling_on_sc=False),
    )(emb, ids_flat)


def _mlp_body(x_ref, w1_ref, b1_ref, w2_ref, b2_ref, o_ref):
    x = x_ref[...]
    h = jnp.dot(x, w1_ref[...], preferred_element_type=jnp.float32)
    h = jnp.maximum(h + b1_ref[...], 0.0)
    o_ref[...] = (
        jnp.dot(h, w2_ref[...], preferred_element_type=jnp.float32)
        + b2_ref[...]
    )


@jax.jit
def _mlp(x, W1, b1, W2, b2):
    blk = 2048
    grid = _B // blk
    return pl.pallas_call(
        _mlp_body,
        grid=(grid,),
        in_specs=[
            pl.BlockSpec((blk, _D), lambda i: (i, 0)),
            pl.BlockSpec((_D, _H), lambda i: (0, 0)),
            pl.BlockSpec((1, _H), lambda i: (0, 0)),
            pl.BlockSpec((_H, _C), lambda i: (0, 0)),
            pl.BlockSpec((1, _C), lambda i: (0, 0)),
        ],
        out_specs=pl.BlockSpec((blk, _C), lambda i: (i, 0)),
        out_shape=jax.ShapeDtypeStruct((_B, _C), jnp.float32),
    )(x, W1, b1.reshape(1, _H), W2, b2.reshape(1, _C))


def kernel(ids, emb, W1, b1, W2, b2):
    ids_flat = ids.reshape(-1).astype(jnp.int32)
    pooled = _pool(ids_flat, emb)
    return _mlp(pooled, W1, b1, W2, b2)
